# baseline (speedup 1.0000x reference)
import jax
import jax.numpy as jnp
from jax.experimental import pallas as pl
from jax.experimental.pallas import tpu as pltpu

_HD = 128
_K_LAT = 32
_KNOTS = 7
_EPS = 1e-4
_ROWS = 16384
_HALF = _ROWS // 2


def _fused_kernel(xk_ref, delta_ref, scale_raw_ref, shift_ref, x_mu_ref,
                  x_std_ref, mu_ref, vk_ref, va_ref, vb_ref, o_ref):
    xk = xk_ref[...]
    seg_dx = xk[1:, :] - xk[:-1, :]
    slopes = jax.nn.softplus(delta_ref[...]) + _EPS
    avg = (jnp.sum(slopes * seg_dx, axis=0, keepdims=True)
           / (jnp.sum(seg_dx, axis=0, keepdims=True) + 1e-8))
    avg = jnp.maximum(avg, 1e-6)
    slopes = slopes / avg

    scale = jax.nn.softplus(scale_raw_ref[...]) + 1e-3
    x_std = x_std_ref[...]
    inv_std = 1.0 / x_std
    a = slopes * (inv_std * scale)
    t = xk * x_std + x_mu_ref[...]
    c = jnp.concatenate([a[0:1, :], a[1:, :] - a[:-1, :]], axis=0)
    const = shift_ref[...] - mu_ref[...] - jnp.sum(c * t[:-1, :], axis=0,
                                                   keepdims=True)

    vk = vk_ref[...]
    bias = jnp.dot(const, vk, preferred_element_type=jnp.float32)
    vk_bf = vk.astype(jnp.bfloat16)

    for v_ref_h, sl in ((va_ref, slice(0, _HALF)),
                        (vb_ref, slice(_HALF, _ROWS))):
        v = v_ref_h[...]
        y = c[0:1, :] * v
        for j in range(1, _KNOTS - 1):
            y = y + c[j:j + 1, :] * jnp.maximum(v, t[j:j + 1, :])
        o_ref[sl, :] = jnp.dot(y.astype(jnp.bfloat16), vk_bf,
                               preferred_element_type=jnp.float32) + bias


def kernel(V, xk, delta_raw, scale_raw, shift, x_mu, x_std, mu, Vk):
    lead = V.shape[:-1]
    n = V.size // _HD
    V2 = V.reshape(n, _HD)
    grid = (n // _ROWS,)

    full = lambda shape: pl.BlockSpec(shape, lambda i: (0,) * len(shape))
    out = pl.pallas_call(
        _fused_kernel,
        grid=grid,
        in_specs=[
            full((_KNOTS, _HD)),
            full((_KNOTS - 1, _HD)),
            full((1, _HD)),
            full((1, _HD)),
            full((1, _HD)),
            full((1, _HD)),
            full((1, _HD)),
            full((_HD, _K_LAT)),
            pl.BlockSpec((_HALF, _HD), lambda i: (2 * i, 0)),
            pl.BlockSpec((_HALF, _HD), lambda i: (2 * i + 1, 0)),
        ],
        out_specs=pl.BlockSpec((_ROWS, _K_LAT), lambda i: (i, 0)),
        out_shape=jax.ShapeDtypeStruct((n, _K_LAT), jnp.float32),
        compiler_params=pltpu.CompilerParams(
            dimension_semantics=("parallel",)),
    )(xk.T, delta_raw.T, scale_raw.reshape(1, _HD), shift.reshape(1, _HD),
      x_mu, x_std, mu, Vk, V2, V2)
    return out.reshape(lead + (_K_LAT,))


# speedup vs baseline: 1.0180x; 1.0180x over previous
import jax
import jax.numpy as jnp
from jax.experimental import pallas as pl
from jax.experimental.pallas import tpu as pltpu

_HD = 128
_K_LAT = 32
_KNOTS = 7
_EPS = 1e-4
_ROWS = 16384


def _fused_kernel(xk_ref, delta_ref, scale_raw_ref, shift_ref, x_mu_ref,
                  x_std_ref, mu_ref, vk_ref, v_ref, o_ref):
    xk = xk_ref[...]
    seg_dx = xk[1:, :] - xk[:-1, :]
    slopes = jax.nn.softplus(delta_ref[...]) + _EPS
    avg = (jnp.sum(slopes * seg_dx, axis=0, keepdims=True)
           / (jnp.sum(seg_dx, axis=0, keepdims=True) + 1e-8))
    avg = jnp.maximum(avg, 1e-6)
    slopes = slopes / avg

    scale = jax.nn.softplus(scale_raw_ref[...]) + 1e-3
    x_std = x_std_ref[...]
    inv_std = 1.0 / x_std
    a = slopes * (inv_std * scale)
    t = xk * x_std + x_mu_ref[...]
    c = jnp.concatenate([a[0:1, :], a[1:, :] - a[:-1, :]], axis=0)
    const = shift_ref[...] - mu_ref[...] - jnp.sum(c * t[:-1, :], axis=0,
                                                   keepdims=True)

    vk = vk_ref[...]
    bias = jnp.dot(const, vk, preferred_element_type=jnp.float32)

    v = v_ref[...]
    y = c[0:1, :] * v
    for j in range(1, _KNOTS - 1):
        y = y + c[j:j + 1, :] * jnp.maximum(v, t[j:j + 1, :])
    o_ref[...] = jnp.dot(y, vk, preferred_element_type=jnp.float32) + bias


def kernel(V, xk, delta_raw, scale_raw, shift, x_mu, x_std, mu, Vk):
    lead = V.shape[:-1]
    n = V.size // _HD
    V2 = V.reshape(n, _HD)
    grid = (n // _ROWS,)

    full = lambda shape: pl.BlockSpec(shape, lambda i: (0,) * len(shape))
    out = pl.pallas_call(
        _fused_kernel,
        grid=grid,
        in_specs=[
            full((_KNOTS, _HD)),
            full((_KNOTS - 1, _HD)),
            full((1, _HD)),
            full((1, _HD)),
            full((1, _HD)),
            full((1, _HD)),
            full((1, _HD)),
            full((_HD, _K_LAT)),
            pl.BlockSpec((_ROWS, _HD), lambda i: (i, 0)),
        ],
        out_specs=pl.BlockSpec((_ROWS, _K_LAT), lambda i: (i, 0)),
        out_shape=jax.ShapeDtypeStruct((n, _K_LAT), jnp.float32),
        compiler_params=pltpu.CompilerParams(
            dimension_semantics=("parallel",)),
    )(xk.T, delta_raw.T, scale_raw.reshape(1, _HD), shift.reshape(1, _HD),
      x_mu, x_std, mu, Vk, V2)
    return out.reshape(lead + (_K_LAT,))
